# baseline (speedup 1.0000x reference)
import functools

import jax
import jax.numpy as jnp
from jax import lax
from jax.experimental import pallas as pl
from jax.experimental.pallas import tpu as pltpu
from jax.experimental.pallas import tpu_sc as plsc

N = 10000
NP = 10240
E = 320000
F = 64
FIN = 128

NC, NS = 2, 16
ROWS_PER_TILE = NP // NS
EDGES_PER_TILE = E // (NC * NS)
NBLK = EDGES_PER_TILE // 128
TAIL = EDGES_PER_TILE - NBLK * 128
NBUF = 3
NGRP = NBLK // (2 * NBUF)

_sc_mesh = plsc.VectorSubcoreMesh(core_axis_name="c", subcore_axis_name="s")

_sc_params = pltpu.CompilerParams(use_tc_tiling_on_sc=False)


def _hist_body(ei_hbm, out_hbm, acc_sh, c0, c1, c2, c3, c4, c5, ctail,
               ones, otail, zbuf, ssem0, ssem1, csem0, csem1):
    c = lax.axis_index("c")
    s = lax.axis_index("s")
    cbufs = (c0, c1, c2, c3, c4, c5)
    ssem = (ssem0, ssem1)
    csem = (csem0, csem1)

    @pl.loop(0, ROWS_PER_TILE // 16)
    def _(i):
        zbuf[pl.ds(i * 16, 16)] = jnp.zeros((16,), jnp.float32)

    @pl.loop(0, 8)
    def _(i):
        ones[pl.ds(i * 16, 16)] = jnp.ones((16,), jnp.float32)

    otail[pl.ds(0, 16)] = jnp.ones((16,), jnp.float32)

    r0 = s * ROWS_PER_TILE
    pltpu.sync_copy(zbuf, acc_sh.at[pl.ds(r0, ROWS_PER_TILE)])
    plsc.subcore_barrier()

    e0 = (c * NS + s) * EDGES_PER_TILE

    def cidx_start(blk, buf, bank):
        pltpu.async_copy(ei_hbm.at[1, pl.ds(e0 + blk * 128, 128)], buf,
                         csem[bank])

    def cidx_wait(blk, buf, bank):
        pltpu.make_async_copy(ei_hbm.at[1, pl.ds(e0 + blk * 128, 128)], buf,
                              csem[bank]).wait()

    def scat_start(buf, bank):
        pltpu.async_copy(ones, acc_sh.at[buf], ssem[bank], add=True)

    def scat_wait(buf, bank):
        pltpu.make_async_copy(ones, acc_sh.at[buf], ssem[bank]).wait()

    @pl.loop(0, NGRP)
    def _(g):
        for bank in range(2):
            base = g * 2 * NBUF + bank * NBUF
            bufs = cbufs[bank * NBUF:bank * NBUF + NBUF]

            @pl.when(g > 0)
            def _():
                for q in range(NBUF):
                    scat_wait(bufs[q], bank)

            for q in range(NBUF):
                cidx_start(base + q, bufs[q], bank)
            for q in range(NBUF):
                cidx_wait(base + q, bufs[q], bank)
            for q in range(NBUF):
                scat_start(bufs[q], bank)

    for bank in range(2):
        for q in range(NBUF):
            scat_wait(cbufs[bank * NBUF + q], bank)

    pltpu.sync_copy(ei_hbm.at[1, pl.ds(e0 + NBLK * 128, TAIL)], ctail)
    pltpu.sync_copy(otail, acc_sh.at[ctail], add=True)

    plsc.subcore_barrier()
    pltpu.sync_copy(acc_sh.at[pl.ds(r0, ROWS_PER_TILE)],
                    out_hbm.at[c, pl.ds(r0, ROWS_PER_TILE)])


@jax.jit
def _sc_histogram(ei):
    kern = pl.kernel(
        _hist_body,
        out_type=jax.ShapeDtypeStruct((NC, NP), jnp.float32),
        mesh=_sc_mesh,
        scratch_types=[pltpu.VMEM_SHARED((NP,), jnp.float32)]
        + [pltpu.VMEM((128,), jnp.int32) for _ in range(6)]
        + [
            pltpu.VMEM((TAIL,), jnp.int32),
            pltpu.VMEM((128,), jnp.float32),
            pltpu.VMEM((TAIL,), jnp.float32),
            pltpu.VMEM((ROWS_PER_TILE,), jnp.float32),
            pltpu.SemaphoreType.DMA,
            pltpu.SemaphoreType.DMA,
            pltpu.SemaphoreType.DMA,
            pltpu.SemaphoreType.DMA,
        ],
        compiler_params=_sc_params,
    )
    return kern(ei)


def _agg_body(src_hbm, ei_hbm, out_hbm, acc_sh, ridx,
              c0, c1, c2, c3, c4, c5, ctail,
              r0b, r1b, r2b, r3b, r4b, r5b, rtail,
              gsem, ssem0, ssem1, csem0, csem1, wsem):
    c = lax.axis_index("c")
    s = lax.axis_index("s")
    cbufs = (c0, c1, c2, c3, c4, c5)
    rows = (r0b, r1b, r2b, r3b, r4b, r5b)
    ssem = (ssem0, ssem1)
    csem = (csem0, csem1)

    @pl.loop(0, 128)
    def _(i):
        @pl.loop(0, F // 16)
        def _(j):
            r0b[i, pl.ds(j * 16, 16)] = jnp.zeros((16,), jnp.float32)

    r0 = s * ROWS_PER_TILE

    @pl.loop(0, ROWS_PER_TILE // 128)
    def _(i):
        pltpu.sync_copy(r0b, acc_sh.at[pl.ds(r0 + i * 128, 128)])

    e0 = (c * NS + s) * EDGES_PER_TILE
    pltpu.sync_copy(ei_hbm.at[0, pl.ds(e0, EDGES_PER_TILE)], ridx)
    plsc.subcore_barrier()

    def cidx_start(blk, buf, bank):
        pltpu.async_copy(ei_hbm.at[1, pl.ds(e0 + blk * 128, 128)], buf,
                         csem[bank])

    def cidx_wait(blk, buf, bank):
        pltpu.make_async_copy(ei_hbm.at[1, pl.ds(e0 + blk * 128, 128)], buf,
                              csem[bank]).wait()

    def gath(blk, buf):
        return pltpu.make_async_copy(
            src_hbm.at[ridx.at[pl.ds(blk * 128, 128)]], buf, gsem)

    def scat_start(cb, buf, bank):
        pltpu.async_copy(buf, acc_sh.at[cb], ssem[bank], add=True)

    def scat_wait(cb, buf, bank):
        pltpu.make_async_copy(buf, acc_sh.at[cb], ssem[bank]).wait()

    @pl.loop(0, NGRP)
    def _(g):
        for bank in range(2):
            base = g * 2 * NBUF + bank * NBUF
            cb = cbufs[bank * NBUF:bank * NBUF + NBUF]
            rb = rows[bank * NBUF:bank * NBUF + NBUF]

            @pl.when(g > 0)
            def _():
                for q in range(NBUF):
                    scat_wait(cb[q], rb[q], bank)

            for q in range(NBUF):
                cidx_start(base + q, cb[q], bank)
                gath(base + q, rb[q]).start()
            for q in range(NBUF):
                gath(base + q, rb[q]).wait()
                cidx_wait(base + q, cb[q], bank)
            for q in range(NBUF):
                scat_start(cb[q], rb[q], bank)

    for bank in range(2):
        for q in range(NBUF):
            scat_wait(cbufs[bank * NBUF + q], rows[bank * NBUF + q], bank)

    pltpu.sync_copy(ei_hbm.at[1, pl.ds(e0 + NBLK * 128, TAIL)], ctail)
    pltpu.sync_copy(src_hbm.at[ridx.at[pl.ds(NBLK * 128, TAIL)]], rtail)
    pltpu.sync_copy(rtail, acc_sh.at[ctail], add=True)

    plsc.subcore_barrier()

    t0 = r0 // 8

    @pl.loop(0, ROWS_PER_TILE // 8)
    def _(t):
        pltpu.async_copy(acc_sh.at[pl.ds(r0 + t * 8, 8)],
                         out_hbm.at[c, t0 + t, :, pl.ds(0, F)], wsem)

    @pl.loop(0, ROWS_PER_TILE // 8)
    def _(t):
        pltpu.make_async_copy(acc_sh.at[pl.ds(r0 + t * 8, 8)],
                              out_hbm.at[c, t0 + t, :, pl.ds(0, F)],
                              wsem).wait()


@jax.jit
def _sc_aggregate(src, ei):
    kern = pl.kernel(
        _agg_body,
        out_type=jax.ShapeDtypeStruct((NC, NP // 8, 8, 128), jnp.float32),
        mesh=_sc_mesh,
        scratch_types=[
            pltpu.VMEM_SHARED((NP, F), jnp.float32),
            pltpu.VMEM((EDGES_PER_TILE,), jnp.int32),
        ]
        + [pltpu.VMEM((128,), jnp.int32) for _ in range(6)]
        + [pltpu.VMEM((TAIL,), jnp.int32)]
        + [pltpu.VMEM((128, F), jnp.float32) for _ in range(6)]
        + [
            pltpu.VMEM((TAIL, F), jnp.float32),
            pltpu.SemaphoreType.DMA,
            pltpu.SemaphoreType.DMA,
            pltpu.SemaphoreType.DMA,
            pltpu.SemaphoreType.DMA,
            pltpu.SemaphoreType.DMA,
            pltpu.SemaphoreType.DMA,
        ],
        compiler_params=_sc_params,
    )
    return kern(src, ei)


BR = 1024
BRN = 1000

_tc_params = pltpu.CompilerParams(dimension_semantics=("parallel",))


def _dinv_of(c_ref):
    return jax.lax.rsqrt(1.0 + c_ref[0] + c_ref[1])[:, None]


def _mm_body(x_ref, w_ref, o_ref):
    o_ref[...] = jnp.dot(x_ref[...], w_ref[...],
                         preferred_element_type=jnp.float32)


def _tc_matmul(x, w):
    m, k = x.shape
    n = w.shape[1]
    return pl.pallas_call(
        _mm_body,
        grid=(m // BRN,),
        in_specs=[
            pl.BlockSpec((BRN, k), lambda i: (i, 0)),
            pl.BlockSpec((k, n), lambda i: (0, 0)),
        ],
        out_specs=pl.BlockSpec((BRN, n), lambda i: (i, 0)),
        out_shape=jax.ShapeDtypeStruct((NP, n), jnp.float32),
        compiler_params=_tc_params,
    )(x, w)


def _scale_body(c_ref, h_ref, o_ref):
    o_ref[...] = _dinv_of(c_ref) * h_ref[...]


def _tc_scale(counts, h):
    m, n = h.shape
    return pl.pallas_call(
        _scale_body,
        grid=(m // BR,),
        in_specs=[
            pl.BlockSpec((NC, BR), lambda i: (0, i)),
            pl.BlockSpec((BR, n), lambda i: (i, 0)),
        ],
        out_specs=pl.BlockSpec((BR, n), lambda i: (i, 0)),
        out_shape=jax.ShapeDtypeStruct((m, n), jnp.float32),
        compiler_params=_tc_params,
    )(counts, h)


def _parts_sum(p_ref):
    nrow = p_ref.shape[1] * 8
    p0 = p_ref[0].reshape(nrow, 128)[:, :F]
    p1 = p_ref[1].reshape(nrow, 128)[:, :F]
    return p0 + p1


def _mid_body(c_ref, p_ref, h_ref, b_ref, w_ref, o_ref):
    dinv = _dinv_of(c_ref)
    agg = _parts_sum(p_ref) + h_ref[...]
    z = jnp.maximum(dinv * agg + b_ref[...], 0.0)
    o_ref[...] = dinv * jnp.dot(z, w_ref[...],
                                preferred_element_type=jnp.float32)


def _tc_mid(counts, parts, hp, b, w):
    m, n = hp.shape
    return pl.pallas_call(
        _mid_body,
        grid=(m // BR,),
        in_specs=[
            pl.BlockSpec((NC, BR), lambda i: (0, i)),
            pl.BlockSpec((NC, BR // 8, 8, 128), lambda i: (0, i, 0, 0)),
            pl.BlockSpec((BR, n), lambda i: (i, 0)),
            pl.BlockSpec((1, n), lambda i: (0, 0)),
            pl.BlockSpec((n, n), lambda i: (0, 0)),
        ],
        out_specs=pl.BlockSpec((BR, n), lambda i: (i, 0)),
        out_shape=jax.ShapeDtypeStruct((m, n), jnp.float32),
        compiler_params=_tc_params,
    )(counts, parts, hp, b, w)


def _out_body(c_ref, p_ref, h_ref, b_ref, w_ref, b3_ref, o_ref):
    dinv = _dinv_of(c_ref)
    agg = _parts_sum(p_ref) + h_ref[...]
    z = jnp.maximum(dinv * agg + b_ref[...], 0.0)
    o_ref[...] = jnp.dot(z, w_ref[...],
                         preferred_element_type=jnp.float32) + b3_ref[...]


def _tc_out(counts, parts, hp, b, w3, b3):
    m, n = hp.shape
    nout = w3.shape[1]
    return pl.pallas_call(
        _out_body,
        grid=(m // BR,),
        in_specs=[
            pl.BlockSpec((NC, BR), lambda i: (0, i)),
            pl.BlockSpec((NC, BR // 8, 8, 128), lambda i: (0, i, 0, 0)),
            pl.BlockSpec((BR, n), lambda i: (i, 0)),
            pl.BlockSpec((1, n), lambda i: (0, 0)),
            pl.BlockSpec((n, nout), lambda i: (0, 0)),
            pl.BlockSpec((1, nout), lambda i: (0, 0)),
        ],
        out_specs=pl.BlockSpec((BR, nout), lambda i: (i, 0)),
        out_shape=jax.ShapeDtypeStruct((m, nout), jnp.float32),
        compiler_params=_tc_params,
    )(counts, parts, hp, b, w3, b3)


@jax.jit
def kernel(x, edge_index, W1, b1, W2, b2, W3, b3):
    h1raw = _tc_matmul(x, W1)
    counts = _sc_histogram(edge_index)

    h1p = _tc_scale(counts, h1raw)
    parts1 = _sc_aggregate(h1p, edge_index)
    h2p = _tc_mid(counts, parts1, h1p, b1.reshape(1, F), W2)
    parts2 = _sc_aggregate(h2p, edge_index)
    outp = _tc_out(counts, parts2, h2p, b2.reshape(1, F), W3,
                   b3.reshape(1, 1))
    return outp[:N]


# speedup vs baseline: 1.0331x; 1.0331x over previous
import functools

import jax
import jax.numpy as jnp
from jax import lax
from jax.experimental import pallas as pl
from jax.experimental.pallas import tpu as pltpu
from jax.experimental.pallas import tpu_sc as plsc

N = 10000
NP = 10240
E = 320000
F = 64
FIN = 128

NC, NS = 2, 16
ROWS_PER_TILE = NP // NS
EDGES_PER_TILE = E // (NC * NS)
NBLK = EDGES_PER_TILE // 128
TAIL = EDGES_PER_TILE - NBLK * 128
NBUF = 3
NGRP = NBLK // (2 * NBUF)

_sc_mesh = plsc.VectorSubcoreMesh(core_axis_name="c", subcore_axis_name="s")

_sc_params = pltpu.CompilerParams(use_tc_tiling_on_sc=False)


def _hist_body(ei_hbm, out_hbm, acc_sh, c0, c1, c2, c3, c4, c5, ctail,
               ones, otail, zbuf, ssem0, ssem1, csem0, csem1):
    c = lax.axis_index("c")
    s = lax.axis_index("s")
    cbufs = (c0, c1, c2, c3, c4, c5)
    ssem = (ssem0, ssem1)
    csem = (csem0, csem1)

    @pl.loop(0, ROWS_PER_TILE // 16)
    def _(i):
        zbuf[pl.ds(i * 16, 16)] = jnp.zeros((16,), jnp.float32)

    @pl.loop(0, 8)
    def _(i):
        ones[pl.ds(i * 16, 16)] = jnp.ones((16,), jnp.float32)

    otail[pl.ds(0, 16)] = jnp.ones((16,), jnp.float32)

    r0 = s * ROWS_PER_TILE
    pltpu.sync_copy(zbuf, acc_sh.at[pl.ds(r0, ROWS_PER_TILE)])
    plsc.subcore_barrier()

    e0 = (c * NS + s) * EDGES_PER_TILE

    def cidx_start(blk, buf, bank):
        pltpu.async_copy(ei_hbm.at[1, pl.ds(e0 + blk * 128, 128)], buf,
                         csem[bank])

    def cidx_wait(blk, buf, bank):
        pltpu.make_async_copy(ei_hbm.at[1, pl.ds(e0 + blk * 128, 128)], buf,
                              csem[bank]).wait()

    def scat_start(buf, bank):
        pltpu.async_copy(ones, acc_sh.at[buf], ssem[bank], add=True)

    def scat_wait(buf, bank):
        pltpu.make_async_copy(ones, acc_sh.at[buf], ssem[bank]).wait()

    @pl.loop(0, NGRP)
    def _(g):
        for bank in range(2):
            base = g * 2 * NBUF + bank * NBUF
            bufs = cbufs[bank * NBUF:bank * NBUF + NBUF]

            @pl.when(g > 0)
            def _():
                for q in range(NBUF):
                    scat_wait(bufs[q], bank)

            for q in range(NBUF):
                cidx_start(base + q, bufs[q], bank)
            for q in range(NBUF):
                cidx_wait(base + q, bufs[q], bank)
            for q in range(NBUF):
                scat_start(bufs[q], bank)

    for bank in range(2):
        for q in range(NBUF):
            scat_wait(cbufs[bank * NBUF + q], bank)

    pltpu.sync_copy(ei_hbm.at[1, pl.ds(e0 + NBLK * 128, TAIL)], ctail)
    pltpu.sync_copy(otail, acc_sh.at[ctail], add=True)

    plsc.subcore_barrier()
    pltpu.sync_copy(acc_sh.at[pl.ds(r0, ROWS_PER_TILE)],
                    out_hbm.at[c, pl.ds(r0, ROWS_PER_TILE)])


@jax.jit
def _sc_histogram(ei):
    kern = pl.kernel(
        _hist_body,
        out_type=jax.ShapeDtypeStruct((NC, NP), jnp.float32),
        mesh=_sc_mesh,
        scratch_types=[pltpu.VMEM_SHARED((NP,), jnp.float32)]
        + [pltpu.VMEM((128,), jnp.int32) for _ in range(6)]
        + [
            pltpu.VMEM((TAIL,), jnp.int32),
            pltpu.VMEM((128,), jnp.float32),
            pltpu.VMEM((TAIL,), jnp.float32),
            pltpu.VMEM((ROWS_PER_TILE,), jnp.float32),
            pltpu.SemaphoreType.DMA,
            pltpu.SemaphoreType.DMA,
            pltpu.SemaphoreType.DMA,
            pltpu.SemaphoreType.DMA,
        ],
        compiler_params=_sc_params,
    )
    return kern(ei)


def _agg_body(src_hbm, ei_hbm, out_hbm, acc_sh, ridx,
              c0, c1, c2, c3, c4, c5, ctail,
              r0b, r1b, r2b, r3b, r4b, r5b, rtail,
              gsem, ssem0, ssem1, csem0, csem1, wsem):
    c = lax.axis_index("c")
    s = lax.axis_index("s")
    cbufs = (c0, c1, c2, c3, c4, c5)
    rows = (r0b, r1b, r2b, r3b, r4b, r5b)
    ssem = (ssem0, ssem1)
    csem = (csem0, csem1)

    @pl.loop(0, 128)
    def _(i):
        @pl.loop(0, F // 16)
        def _(j):
            r0b[i, pl.ds(j * 16, 16)] = jnp.zeros((16,), jnp.float32)

    r0 = s * ROWS_PER_TILE

    @pl.loop(0, ROWS_PER_TILE // 128)
    def _(i):
        pltpu.sync_copy(r0b, acc_sh.at[pl.ds(r0 + i * 128, 128)])

    e0 = (c * NS + s) * EDGES_PER_TILE
    pltpu.sync_copy(ei_hbm.at[0, pl.ds(e0, EDGES_PER_TILE)], ridx)
    plsc.subcore_barrier()

    def cidx_start(blk, buf, bank):
        pltpu.async_copy(ei_hbm.at[1, pl.ds(e0 + blk * 128, 128)], buf,
                         csem[bank])

    def cidx_wait(blk, buf, bank):
        pltpu.make_async_copy(ei_hbm.at[1, pl.ds(e0 + blk * 128, 128)], buf,
                              csem[bank]).wait()

    def gath(blk, buf):
        return pltpu.make_async_copy(
            src_hbm.at[ridx.at[pl.ds(blk * 128, 128)]], buf, gsem)

    def scat_start(cb, buf, bank):
        pltpu.async_copy(buf, acc_sh.at[cb], ssem[bank], add=True)

    def scat_wait(cb, buf, bank):
        pltpu.make_async_copy(buf, acc_sh.at[cb], ssem[bank]).wait()

    @pl.loop(0, NGRP)
    def _(g):
        for bank in range(2):
            base = g * 2 * NBUF + bank * NBUF
            cb = cbufs[bank * NBUF:bank * NBUF + NBUF]
            rb = rows[bank * NBUF:bank * NBUF + NBUF]

            @pl.when(g > 0)
            def _():
                for q in range(NBUF):
                    scat_wait(cb[q], rb[q], bank)

            for q in range(NBUF):
                cidx_start(base + q, cb[q], bank)
                gath(base + q, rb[q]).start()
            for q in range(NBUF):
                gath(base + q, rb[q]).wait()
                cidx_wait(base + q, cb[q], bank)
            for q in range(NBUF):
                scat_start(cb[q], rb[q], bank)

    for bank in range(2):
        for q in range(NBUF):
            scat_wait(cbufs[bank * NBUF + q], rows[bank * NBUF + q], bank)

    pltpu.sync_copy(ei_hbm.at[1, pl.ds(e0 + NBLK * 128, TAIL)], ctail)
    pltpu.sync_copy(src_hbm.at[ridx.at[pl.ds(NBLK * 128, TAIL)]], rtail)
    pltpu.sync_copy(rtail, acc_sh.at[ctail], add=True)

    plsc.subcore_barrier()

    t0 = r0 // 8

    @pl.loop(0, ROWS_PER_TILE // 8)
    def _(t):
        pltpu.async_copy(acc_sh.at[pl.ds(r0 + t * 8, 8)],
                         out_hbm.at[c, t0 + t, :, pl.ds(0, F)], wsem)

    @pl.loop(0, ROWS_PER_TILE // 8)
    def _(t):
        pltpu.make_async_copy(acc_sh.at[pl.ds(r0 + t * 8, 8)],
                              out_hbm.at[c, t0 + t, :, pl.ds(0, F)],
                              wsem).wait()


@jax.jit
def _sc_aggregate(src, ei):
    kern = pl.kernel(
        _agg_body,
        out_type=jax.ShapeDtypeStruct((NC, NP // 8, 8, 128), jnp.float32),
        mesh=_sc_mesh,
        scratch_types=[
            pltpu.VMEM_SHARED((NP, F), jnp.float32),
            pltpu.VMEM((EDGES_PER_TILE,), jnp.int32),
        ]
        + [pltpu.VMEM((128,), jnp.int32) for _ in range(6)]
        + [pltpu.VMEM((TAIL,), jnp.int32)]
        + [pltpu.VMEM((128, F), jnp.float32) for _ in range(6)]
        + [
            pltpu.VMEM((TAIL, F), jnp.float32),
            pltpu.SemaphoreType.DMA,
            pltpu.SemaphoreType.DMA,
            pltpu.SemaphoreType.DMA,
            pltpu.SemaphoreType.DMA,
            pltpu.SemaphoreType.DMA,
            pltpu.SemaphoreType.DMA,
        ],
        compiler_params=_sc_params,
    )
    return kern(src, ei)


BR = 2048
BRN = 2000

_tc_params = pltpu.CompilerParams(dimension_semantics=("parallel",))


def _dinv_of(c_ref):
    return jax.lax.rsqrt(1.0 + c_ref[0] + c_ref[1])[:, None]


def _mm_body(x_ref, w_ref, o_ref):
    o_ref[...] = jnp.dot(x_ref[...], w_ref[...],
                         preferred_element_type=jnp.float32)


def _tc_matmul(x, w):
    m, k = x.shape
    n = w.shape[1]
    return pl.pallas_call(
        _mm_body,
        grid=(m // BRN,),
        in_specs=[
            pl.BlockSpec((BRN, k), lambda i: (i, 0)),
            pl.BlockSpec((k, n), lambda i: (0, 0)),
        ],
        out_specs=pl.BlockSpec((BRN, n), lambda i: (i, 0)),
        out_shape=jax.ShapeDtypeStruct((NP, n), jnp.float32),
        compiler_params=_tc_params,
    )(x, w)


def _scale_body(c_ref, h_ref, o_ref):
    o_ref[...] = _dinv_of(c_ref) * h_ref[...]


def _tc_scale(counts, h):
    m, n = h.shape
    return pl.pallas_call(
        _scale_body,
        grid=(m // BR,),
        in_specs=[
            pl.BlockSpec((NC, BR), lambda i: (0, i)),
            pl.BlockSpec((BR, n), lambda i: (i, 0)),
        ],
        out_specs=pl.BlockSpec((BR, n), lambda i: (i, 0)),
        out_shape=jax.ShapeDtypeStruct((m, n), jnp.float32),
        compiler_params=_tc_params,
    )(counts, h)


def _parts_sum(p_ref):
    nrow = p_ref.shape[1] * 8
    p0 = p_ref[0].reshape(nrow, 128)[:, :F]
    p1 = p_ref[1].reshape(nrow, 128)[:, :F]
    return p0 + p1


def _mid_body(c_ref, p_ref, h_ref, b_ref, w_ref, o_ref):
    dinv = _dinv_of(c_ref)
    agg = _parts_sum(p_ref) + h_ref[...]
    z = jnp.maximum(dinv * agg + b_ref[...], 0.0)
    o_ref[...] = dinv * jnp.dot(z, w_ref[...],
                                preferred_element_type=jnp.float32)


def _tc_mid(counts, parts, hp, b, w):
    m, n = hp.shape
    return pl.pallas_call(
        _mid_body,
        grid=(m // BR,),
        in_specs=[
            pl.BlockSpec((NC, BR), lambda i: (0, i)),
            pl.BlockSpec((NC, BR // 8, 8, 128), lambda i: (0, i, 0, 0)),
            pl.BlockSpec((BR, n), lambda i: (i, 0)),
            pl.BlockSpec((1, n), lambda i: (0, 0)),
            pl.BlockSpec((n, n), lambda i: (0, 0)),
        ],
        out_specs=pl.BlockSpec((BR, n), lambda i: (i, 0)),
        out_shape=jax.ShapeDtypeStruct((m, n), jnp.float32),
        compiler_params=_tc_params,
    )(counts, parts, hp, b, w)


def _out_body(c_ref, p_ref, h_ref, b_ref, w_ref, b3_ref, o_ref):
    dinv = _dinv_of(c_ref)
    agg = _parts_sum(p_ref) + h_ref[...]
    z = jnp.maximum(dinv * agg + b_ref[...], 0.0)
    o_ref[...] = jnp.dot(z, w_ref[...],
                         preferred_element_type=jnp.float32) + b3_ref[...]


def _tc_out(counts, parts, hp, b, w3, b3):
    m, n = hp.shape
    nout = w3.shape[1]
    return pl.pallas_call(
        _out_body,
        grid=(m // BR,),
        in_specs=[
            pl.BlockSpec((NC, BR), lambda i: (0, i)),
            pl.BlockSpec((NC, BR // 8, 8, 128), lambda i: (0, i, 0, 0)),
            pl.BlockSpec((BR, n), lambda i: (i, 0)),
            pl.BlockSpec((1, n), lambda i: (0, 0)),
            pl.BlockSpec((n, nout), lambda i: (0, 0)),
            pl.BlockSpec((1, nout), lambda i: (0, 0)),
        ],
        out_specs=pl.BlockSpec((BR, nout), lambda i: (i, 0)),
        out_shape=jax.ShapeDtypeStruct((m, nout), jnp.float32),
        compiler_params=_tc_params,
    )(counts, parts, hp, b, w3, b3)


@jax.jit
def kernel(x, edge_index, W1, b1, W2, b2, W3, b3):
    h1raw = _tc_matmul(x, W1)
    counts = _sc_histogram(edge_index)

    h1p = _tc_scale(counts, h1raw)
    parts1 = _sc_aggregate(h1p, edge_index)
    h2p = _tc_mid(counts, parts1, h1p, b1.reshape(1, F), W2)
    parts2 = _sc_aggregate(h2p, edge_index)
    outp = _tc_out(counts, parts2, h2p, b2.reshape(1, F), W3,
                   b3.reshape(1, 1))
    return outp[:N]
